# baseline (speedup 1.0000x reference)
import jax
import jax.numpy as jnp
from jax import lax
from jax.experimental import pallas as pl
from jax.experimental.pallas import tpu as pltpu
from jax.experimental.pallas import tpu_sc as plsc

N = 2_000_000
G = 50_000
P = 500_000
SUB = 128
NSUB = 5
B = SUB * NSUB
NCHUNK = N // B
L = 16
NC = 2
NW = 32
TW = 8

GB = 3136
G_PAD = 16 * GB
PT = 31_280
P_PAD = 16 * PT
PB = 1360


def _cross(ax, ay, az, bx, by, bz):
    return ay * bz - az * by, az * bx - ax * bz, ax * by - ay * bx


def _rotate_conj(qx, qy, qz, qw, vx, vy, vz):
    tx, ty, tz = _cross(vx, vy, vz, qx, qy, qz)
    tx, ty, tz = 2.0 * tx, 2.0 * ty, 2.0 * tz
    cx, cy, cz = _cross(tx, ty, tz, qx, qy, qz)
    return vx + qw * tx + cx, vy + qw * ty + cy, vz + qw * tz + cz


def _sc_kernel(fx_hbm, fy_hbm, fz_hbm, gidx_hbm, midx_hbm, pidx_hbm,
               scales_hbm, relf_hbm, *rest):
    gcomp_hbm = rest[0:8]
    pcomp_hbm = rest[8:11]
    ox_hbm, oy_hbm, oz_hbm = rest[11:14]
    scr = rest[14:]

    NS = 16
    sets = []
    for p in range(2):
        s = scr[p * NS:(p + 1) * NS]
        sets.append(dict(
            gidx=s[0], pidx=s[1], midx=s[2], scales=s[3],
            fx=s[4], fy=s[5], fz=s[6],
            gtab=s[7], ptab=s[8], ox=s[9], oy=s[10], oz=s[11],
            sem_idx=s[12], sem_feat=s[13], sem_gat=s[14], sem_out=s[15]))
    (gtab_s, ptab_s, bld_v, relf_v, ax_v, ay_v, az_v, aw_v,
     ux_v, uy_v, uz_v, sem_bld) = scr[2 * NS:2 * NS + 12]
    stg = scr[2 * NS + 12:2 * NS + 20]
    S0, S1 = sets

    cc = lax.axis_index("c")
    s16 = lax.axis_index("s")
    wid = s16 * NC + cc
    nchunk = fx_hbm.shape[0] // B
    niter = (nchunk - wid + NW - 1) // NW
    iota = lax.iota(jnp.int32, L)
    cols = [jnp.full((L,), t, jnp.int32) for t in range(TW)]

    mac = pltpu.make_async_copy

    def fire(copies):
        for d in copies:
            d.start()

    def drain(copies):
        for d in copies:
            d.wait()

    mytab_g = gtab_s.at[cc]
    mytab_p = ptab_s.at[cc]

    start_g = s16 * GB
    bs = [mac(gcomp_hbm[t].at[pl.ds(start_g, GB)], stg[t], sem_bld)
          for t in range(8)]
    fire(bs)
    drain(bs)

    def gb_body(i, carry):
        b16 = i * L
        lids = b16 + iota
        for t in range(TW):
            plsc.store_scatter(bld_v, [lids, cols[t]], stg[t][pl.ds(b16, L)])
        return carry

    lax.fori_loop(0, GB // L, gb_body, 0)
    d = mac(bld_v, mytab_g.at[pl.ds(start_g, GB)], sem_bld)
    d.start()
    d.wait()

    def pb_outer(j, carry):
        start_p = s16 * PT + j * PB
        bs = [mac(pcomp_hbm[t].at[pl.ds(start_p, PB)],
                  stg[t].at[pl.ds(0, PB)], sem_bld) for t in range(3)]
        fire(bs)
        drain(bs)

        def pb_body(i, carry2):
            b16 = i * L
            lids = b16 + iota
            for t in range(3):
                plsc.store_scatter(bld_v, [lids, cols[t]],
                                   stg[t][pl.ds(b16, L)])
            return carry2

        lax.fori_loop(0, PB // L, pb_body, 0)
        d = mac(bld_v.at[pl.ds(0, PB)], mytab_p.at[pl.ds(start_p, PB)], sem_bld)
        d.start()
        d.wait()
        return carry

    lax.fori_loop(0, PT // PB, pb_outer, 0)
    plsc.subcore_barrier()

    pltpu.sync_copy(relf_hbm, relf_v)
    i4 = iota * 4
    rax = plsc.load_gather(relf_v, [i4])
    ray = plsc.load_gather(relf_v, [i4 + 1])
    raz = plsc.load_gather(relf_v, [i4 + 2])
    raw = plsc.load_gather(relf_v, [i4 + 3])
    i3 = iota * 3 + 64
    rtx = plsc.load_gather(relf_v, [i3])
    rty = plsc.load_gather(relf_v, [i3 + 1])
    rtz = plsc.load_gather(relf_v, [i3 + 2])
    ux, uy, uz = _rotate_conj(rax, ray, raz, raw, rtx, rty, rtz)
    ax_v[...] = rax
    ay_v[...] = ray
    az_v[...] = raz
    aw_v[...] = raw
    ux_v[...] = ux
    uy_v[...] = uy
    uz_v[...] = uz

    iota3 = iota * 3

    def idx_copies(k, S):
        base = (wid + k * NW) * B
        return [
            mac(gidx_hbm.at[pl.ds(base, B)], S['gidx'], S['sem_idx']),
            mac(pidx_hbm.at[pl.ds(base, B)], S['pidx'], S['sem_idx']),
        ]

    def feat_copies(k, S):
        base = (wid + k * NW) * B
        dB = pl.ds(base, B)
        return [
            mac(fx_hbm.at[dB], S['fx'], S['sem_feat']),
            mac(fy_hbm.at[dB], S['fy'], S['sem_feat']),
            mac(fz_hbm.at[dB], S['fz'], S['sem_feat']),
            mac(scales_hbm.at[dB], S['scales'], S['sem_feat']),
            mac(midx_hbm.at[dB], S['midx'], S['sem_feat']),
        ]

    def gat_copies(k, S):
        out = []
        for j in range(NSUB):
            d = pl.ds(j * SUB, SUB)
            out.append(mac(mytab_g.at[S['gidx'].at[d]], S['gtab'].at[d], S['sem_gat']))
            out.append(mac(mytab_p.at[S['pidx'].at[d]], S['ptab'].at[d], S['sem_gat']))
        return out

    def out_copies(k, S):
        base = (wid + k * NW) * B
        dB = pl.ds(base, B)
        return [
            mac(S['ox'], ox_hbm.at[dB], S['sem_out']),
            mac(S['oy'], oy_hbm.at[dB], S['sem_out']),
            mac(S['oz'], oz_hbm.at[dB], S['sem_out']),
        ]

    def compute(S):
        midx_v, scales_v = S['midx'], S['scales']
        fx_v, fy_v, fz_v = S['fx'], S['fy'], S['fz']
        ox_v, oy_v, oz_v = S['ox'], S['oy'], S['oz']
        gtab_v, ptab_v = S['gtab'], S['ptab']

        def group_body(g, carry):
            b16 = g * L
            d16 = pl.ds(b16, L)
            lids = b16 + iota
            m = midx_v[d16]
            s = scales_v[d16]
            bx = plsc.load_gather(gtab_v, [lids, cols[0]])
            by = plsc.load_gather(gtab_v, [lids, cols[1]])
            bz = plsc.load_gather(gtab_v, [lids, cols[2]])
            bw = plsc.load_gather(gtab_v, [lids, cols[3]])
            tx = plsc.load_gather(gtab_v, [lids, cols[4]])
            ty = plsc.load_gather(gtab_v, [lids, cols[5]])
            tz = plsc.load_gather(gtab_v, [lids, cols[6]])
            wgt = plsc.load_gather(gtab_v, [lids, cols[7]])
            px = plsc.load_gather(ptab_v, [lids, cols[0]])
            py = plsc.load_gather(ptab_v, [lids, cols[1]])
            pz = plsc.load_gather(ptab_v, [lids, cols[2]])
            fx, fy, fz = fx_v[d16], fy_v[d16], fz_v[d16]
            ax = plsc.load_gather(ax_v, [m])
            ay = plsc.load_gather(ay_v, [m])
            az = plsc.load_gather(az_v, [m])
            aw = plsc.load_gather(aw_v, [m])
            mux = plsc.load_gather(ux_v, [m])
            muy = plsc.load_gather(uy_v, [m])
            muz = plsc.load_gather(uz_v, [m])

            qw = aw * bw - ax * bx - ay * by - az * bz
            qx = aw * bx + ax * bw + ay * bz - az * by
            qy = aw * by - ax * bz + ay * bw + az * bx
            qz = aw * bz + ax * by - ay * bx + az * bw

            vx, vy, vz = mux + tx, muy + ty, muz + tz
            r1x, r1y, r1z = _rotate_conj(bx, by, bz, bw, vx, vy, vz)
            r2x, r2y, r2z = _rotate_conj(qx, qy, qz, qw, fx, fy, fz)

            ox_v[d16] = wgt * (px + r1x - s * r2x)
            oy_v[d16] = wgt * (py + r1y - s * r2y)
            oz_v[d16] = wgt * (pz + r1z - s * r2z)
            return carry

        lax.fori_loop(0, B // L, group_body, 0)

    def step(k, S, Sn):
        @pl.when(k + 1 < niter)
        def _():
            drain(idx_copies(k + 1, Sn))
            fire(gat_copies(k + 1, Sn))
        drain(gat_copies(k, S))

        @pl.when(k + 2 < niter)
        def _():
            fire(idx_copies(k + 2, S))
        drain(feat_copies(k, S))

        @pl.when(k >= 2)
        def _():
            drain(out_copies(k - 2, S))
        compute(S)
        fire(out_copies(k, S))

        @pl.when(k + 2 < niter)
        def _():
            fire(feat_copies(k + 2, S))

    fire(idx_copies(0, S0))
    fire(feat_copies(0, S0))
    drain(idx_copies(0, S0))
    fire(gat_copies(0, S0))
    fire(idx_copies(1, S1))
    fire(feat_copies(1, S1))

    def pair_body(t, carry):
        k0 = 2 * t
        step(k0, S0, S1)
        step(k0 + 1, S1, S0)
        return carry

    lax.fori_loop(0, niter // 2, pair_body, 0)

    @pl.when(niter % 2 == 1)
    def _():
        step(niter - 1, S0, S1)

    drain(out_copies(0, S0))
    drain(out_copies(0, S1))


def kernel(feature_undist, grouping_indices, point_indices, is_calibrated,
           ref_rots, rel_rots, points_3d, scales, ref_trans, rel_trans):
    gidx = grouping_indices[:, 0]
    midx = grouping_indices[:, 1]
    calw = 0.5 + 0.5 * is_calibrated.astype(jnp.float32)
    scales_flat = scales.reshape(N)
    relf = jnp.concatenate([rel_rots.reshape(64), rel_trans.reshape(48)])

    def padg(x):
        return jnp.pad(x, (0, G_PAD - G))

    def padp(x):
        return jnp.pad(x, (0, P_PAD - P))

    gcomps = [padg(ref_rots[:, 0]), padg(ref_rots[:, 1]), padg(ref_rots[:, 2]),
              padg(ref_rots[:, 3]), padg(ref_trans[:, 0]),
              padg(ref_trans[:, 1]), padg(ref_trans[:, 2]), padg(calw)]
    pcomps = [padp(points_3d[:, 0]), padp(points_3d[:, 1]),
              padp(points_3d[:, 2])]

    mesh = plsc.VectorSubcoreMesh(core_axis_name="c", subcore_axis_name="s")
    f32, i32 = jnp.float32, jnp.int32
    per_set = [
        pltpu.VMEM((B,), i32),
        pltpu.VMEM((B,), i32),
        pltpu.VMEM((B,), i32),
        pltpu.VMEM((B,), f32),
        pltpu.VMEM((B,), f32),
        pltpu.VMEM((B,), f32),
        pltpu.VMEM((B,), f32),
        pltpu.VMEM((B, TW), f32),
        pltpu.VMEM((B, TW), f32),
        pltpu.VMEM((B,), f32),
        pltpu.VMEM((B,), f32),
        pltpu.VMEM((B,), f32),
        pltpu.SemaphoreType.DMA,
        pltpu.SemaphoreType.DMA,
        pltpu.SemaphoreType.DMA,
        pltpu.SemaphoreType.DMA,
    ]
    def make_run(n):
        return pl.kernel(
            _sc_kernel, mesh=mesh,
            out_type=(jax.ShapeDtypeStruct((n,), f32),) * 3,
            compiler_params=pltpu.CompilerParams(
                needs_layout_passes=False, use_tc_tiling_on_sc=False),
            scratch_types=per_set + per_set + [
                pltpu.HBM((NC, G_PAD, TW), f32),
                pltpu.HBM((NC, P_PAD, TW), f32),
                pltpu.VMEM((GB, TW), f32),
                pltpu.VMEM((112,), f32),
                pltpu.VMEM((16,), f32),
                pltpu.VMEM((16,), f32),
                pltpu.VMEM((16,), f32),
                pltpu.VMEM((16,), f32),
                pltpu.VMEM((16,), f32),
                pltpu.VMEM((16,), f32),
                pltpu.VMEM((16,), f32),
                pltpu.SemaphoreType.DMA,
            ] + [pltpu.VMEM((GB,), f32)] * 8,
        )

    H1 = 1563 * B
    outs = []
    for lo, hi in ((0, H1), (H1, N)):
        r = make_run(hi - lo)
        outs.append(r(
            feature_undist[lo:hi, 0], feature_undist[lo:hi, 1],
            feature_undist[lo:hi, 2], gidx[lo:hi], midx[lo:hi],
            point_indices[lo:hi], scales_flat[lo:hi], relf,
            *gcomps, *pcomps))
    h1 = jnp.stack(outs[0], axis=1)
    h2 = jnp.stack(outs[1], axis=1)
    return jnp.concatenate([h1, h2], axis=0)


# speedup vs baseline: 1.0718x; 1.0718x over previous
import jax
import jax.numpy as jnp
from jax import lax
from jax.experimental import pallas as pl
from jax.experimental.pallas import tpu as pltpu
from jax.experimental.pallas import tpu_sc as plsc

N = 2_000_000
G = 50_000
P = 500_000
SUB = 128
NSUB = 5
B = SUB * NSUB
NCHUNK = N // B
L = 16
NC = 2
NW = 32
TW = 8

GB = 3136
G_PAD = 16 * GB
PT = 31_280
P_PAD = 16 * PT
PB = 1360


def _cross(ax, ay, az, bx, by, bz):
    return ay * bz - az * by, az * bx - ax * bz, ax * by - ay * bx


def _rotate_conj(qx, qy, qz, qw, vx, vy, vz):
    tx, ty, tz = _cross(vx, vy, vz, qx, qy, qz)
    tx, ty, tz = 2.0 * tx, 2.0 * ty, 2.0 * tz
    cx, cy, cz = _cross(tx, ty, tz, qx, qy, qz)
    return vx + qw * tx + cx, vy + qw * ty + cy, vz + qw * tz + cz


def _sc_kernel(fx_hbm, fy_hbm, fz_hbm, gidx_hbm, midx_hbm, pidx_hbm,
               scales_hbm, relf_hbm, *rest):
    gcomp_hbm = rest[0:8]
    pcomp_hbm = rest[8:11]
    ox_hbm, oy_hbm, oz_hbm = rest[11:14]
    scr = rest[14:]

    NS = 16
    sets = []
    for p in range(2):
        s = scr[p * NS:(p + 1) * NS]
        sets.append(dict(
            gidx=s[0], pidx=s[1], midx=s[2], scales=s[3],
            fx=s[4], fy=s[5], fz=s[6],
            gtab=s[7], ptab=s[8], ox=s[9], oy=s[10], oz=s[11],
            sem_idx=s[12], sem_feat=s[13], sem_gat=s[14], sem_out=s[15]))
    (gtab_s, ptab_s, bld_v, relf_v, ax_v, ay_v, az_v, aw_v,
     ux_v, uy_v, uz_v, sem_bld) = scr[2 * NS:2 * NS + 12]
    stg = scr[2 * NS + 12:2 * NS + 20]
    S0, S1 = sets

    cc = lax.axis_index("c")
    s16 = lax.axis_index("s")
    wid = s16 * NC + cc
    niter = (NCHUNK - wid + NW - 1) // NW
    iota = lax.iota(jnp.int32, L)
    cols = [jnp.full((L,), t, jnp.int32) for t in range(TW)]

    mac = pltpu.make_async_copy

    def fire(copies):
        for d in copies:
            d.start()

    def drain(copies):
        for d in copies:
            d.wait()

    mytab_g = gtab_s.at[cc]
    mytab_p = ptab_s.at[cc]

    start_g = s16 * GB
    bs = [mac(gcomp_hbm[t].at[pl.ds(start_g, GB)], stg[t], sem_bld)
          for t in range(8)]
    fire(bs)
    drain(bs)

    def gb_body(i, carry):
        b16 = i * L
        lids = b16 + iota
        for t in range(TW):
            plsc.store_scatter(bld_v, [lids, cols[t]], stg[t][pl.ds(b16, L)])
        return carry

    lax.fori_loop(0, GB // L, gb_body, 0)
    d = mac(bld_v, mytab_g.at[pl.ds(start_g, GB)], sem_bld)
    d.start()
    d.wait()

    def pb_outer(j, carry):
        start_p = s16 * PT + j * PB
        bs = [mac(pcomp_hbm[t].at[pl.ds(start_p, PB)],
                  stg[t].at[pl.ds(0, PB)], sem_bld) for t in range(3)]
        fire(bs)
        drain(bs)

        def pb_body(i, carry2):
            b16 = i * L
            lids = b16 + iota
            for t in range(3):
                plsc.store_scatter(bld_v, [lids, cols[t]],
                                   stg[t][pl.ds(b16, L)])
            return carry2

        lax.fori_loop(0, PB // L, pb_body, 0)
        d = mac(bld_v.at[pl.ds(0, PB)], mytab_p.at[pl.ds(start_p, PB)], sem_bld)
        d.start()
        d.wait()
        return carry

    lax.fori_loop(0, PT // PB, pb_outer, 0)
    plsc.subcore_barrier()

    pltpu.sync_copy(relf_hbm, relf_v)
    i4 = iota * 4
    rax = plsc.load_gather(relf_v, [i4])
    ray = plsc.load_gather(relf_v, [i4 + 1])
    raz = plsc.load_gather(relf_v, [i4 + 2])
    raw = plsc.load_gather(relf_v, [i4 + 3])
    i3 = iota * 3 + 64
    rtx = plsc.load_gather(relf_v, [i3])
    rty = plsc.load_gather(relf_v, [i3 + 1])
    rtz = plsc.load_gather(relf_v, [i3 + 2])
    ux, uy, uz = _rotate_conj(rax, ray, raz, raw, rtx, rty, rtz)
    ax_v[...] = rax
    ay_v[...] = ray
    az_v[...] = raz
    aw_v[...] = raw
    ux_v[...] = ux
    uy_v[...] = uy
    uz_v[...] = uz

    iota3 = iota * 3

    def idx_copies(k, S):
        base = (wid + k * NW) * B
        return [
            mac(gidx_hbm.at[pl.ds(base, B)], S['gidx'], S['sem_idx']),
            mac(pidx_hbm.at[pl.ds(base, B)], S['pidx'], S['sem_idx']),
        ]

    def feat_copies(k, S):
        base = (wid + k * NW) * B
        dB = pl.ds(base, B)
        return [
            mac(fx_hbm.at[dB], S['fx'], S['sem_feat']),
            mac(fy_hbm.at[dB], S['fy'], S['sem_feat']),
            mac(fz_hbm.at[dB], S['fz'], S['sem_feat']),
            mac(scales_hbm.at[dB], S['scales'], S['sem_feat']),
            mac(midx_hbm.at[dB], S['midx'], S['sem_feat']),
        ]

    def gat_copies(k, S):
        out = []
        for j in range(NSUB):
            d = pl.ds(j * SUB, SUB)
            out.append(mac(mytab_g.at[S['gidx'].at[d]], S['gtab'].at[d], S['sem_gat']))
            out.append(mac(mytab_p.at[S['pidx'].at[d]], S['ptab'].at[d], S['sem_gat']))
        return out

    def out_copies(k, S):
        base = (wid + k * NW) * B
        dB = pl.ds(base, B)
        return [
            mac(S['ox'], ox_hbm.at[dB], S['sem_out']),
            mac(S['oy'], oy_hbm.at[dB], S['sem_out']),
            mac(S['oz'], oz_hbm.at[dB], S['sem_out']),
        ]

    def compute(S):
        midx_v, scales_v = S['midx'], S['scales']
        fx_v, fy_v, fz_v = S['fx'], S['fy'], S['fz']
        ox_v, oy_v, oz_v = S['ox'], S['oy'], S['oz']
        gtab_v, ptab_v = S['gtab'], S['ptab']

        def group_body(g, carry):
            b16 = g * L
            d16 = pl.ds(b16, L)
            lids = b16 + iota
            m = midx_v[d16]
            s = scales_v[d16]
            bx = plsc.load_gather(gtab_v, [lids, cols[0]])
            by = plsc.load_gather(gtab_v, [lids, cols[1]])
            bz = plsc.load_gather(gtab_v, [lids, cols[2]])
            bw = plsc.load_gather(gtab_v, [lids, cols[3]])
            tx = plsc.load_gather(gtab_v, [lids, cols[4]])
            ty = plsc.load_gather(gtab_v, [lids, cols[5]])
            tz = plsc.load_gather(gtab_v, [lids, cols[6]])
            wgt = plsc.load_gather(gtab_v, [lids, cols[7]])
            px = plsc.load_gather(ptab_v, [lids, cols[0]])
            py = plsc.load_gather(ptab_v, [lids, cols[1]])
            pz = plsc.load_gather(ptab_v, [lids, cols[2]])
            fx, fy, fz = fx_v[d16], fy_v[d16], fz_v[d16]
            ax = plsc.load_gather(ax_v, [m])
            ay = plsc.load_gather(ay_v, [m])
            az = plsc.load_gather(az_v, [m])
            aw = plsc.load_gather(aw_v, [m])
            mux = plsc.load_gather(ux_v, [m])
            muy = plsc.load_gather(uy_v, [m])
            muz = plsc.load_gather(uz_v, [m])

            qw = aw * bw - ax * bx - ay * by - az * bz
            qx = aw * bx + ax * bw + ay * bz - az * by
            qy = aw * by - ax * bz + ay * bw + az * bx
            qz = aw * bz + ax * by - ay * bx + az * bw

            vx, vy, vz = mux + tx, muy + ty, muz + tz
            r1x, r1y, r1z = _rotate_conj(bx, by, bz, bw, vx, vy, vz)
            r2x, r2y, r2z = _rotate_conj(qx, qy, qz, qw, fx, fy, fz)

            ox_v[d16] = wgt * (px + r1x - s * r2x)
            oy_v[d16] = wgt * (py + r1y - s * r2y)
            oz_v[d16] = wgt * (pz + r1z - s * r2z)
            return carry

        lax.fori_loop(0, B // L, group_body, 0)

    def step(k, S, Sn):
        @pl.when(k + 1 < niter)
        def _():
            drain(idx_copies(k + 1, Sn))
            fire(gat_copies(k + 1, Sn))
        drain(gat_copies(k, S))

        @pl.when(k + 2 < niter)
        def _():
            fire(idx_copies(k + 2, S))
        drain(feat_copies(k, S))

        @pl.when(k >= 2)
        def _():
            drain(out_copies(k - 2, S))
        compute(S)
        fire(out_copies(k, S))

        @pl.when(k + 2 < niter)
        def _():
            fire(feat_copies(k + 2, S))

    fire(idx_copies(0, S0))
    fire(feat_copies(0, S0))
    drain(idx_copies(0, S0))
    fire(gat_copies(0, S0))
    fire(idx_copies(1, S1))
    fire(feat_copies(1, S1))

    def pair_body(t, carry):
        k0 = 2 * t
        step(k0, S0, S1)
        step(k0 + 1, S1, S0)
        return carry

    lax.fori_loop(0, niter // 2, pair_body, 0)

    @pl.when(niter % 2 == 1)
    def _():
        step(niter - 1, S0, S1)

    drain(out_copies(0, S0))
    drain(out_copies(0, S1))


def kernel(feature_undist, grouping_indices, point_indices, is_calibrated,
           ref_rots, rel_rots, points_3d, scales, ref_trans, rel_trans):
    gidx = grouping_indices[:, 0]
    midx = grouping_indices[:, 1]
    calw = 0.5 + 0.5 * is_calibrated.astype(jnp.float32)
    scales_flat = scales.reshape(N)
    relf = jnp.concatenate([rel_rots.reshape(64), rel_trans.reshape(48)])

    def padg(x):
        return jnp.pad(x, (0, G_PAD - G))

    def padp(x):
        return jnp.pad(x, (0, P_PAD - P))

    gcomps = [padg(ref_rots[:, 0]), padg(ref_rots[:, 1]), padg(ref_rots[:, 2]),
              padg(ref_rots[:, 3]), padg(ref_trans[:, 0]),
              padg(ref_trans[:, 1]), padg(ref_trans[:, 2]), padg(calw)]
    pcomps = [padp(points_3d[:, 0]), padp(points_3d[:, 1]),
              padp(points_3d[:, 2])]

    mesh = plsc.VectorSubcoreMesh(core_axis_name="c", subcore_axis_name="s")
    f32, i32 = jnp.float32, jnp.int32
    per_set = [
        pltpu.VMEM((B,), i32),
        pltpu.VMEM((B,), i32),
        pltpu.VMEM((B,), i32),
        pltpu.VMEM((B,), f32),
        pltpu.VMEM((B,), f32),
        pltpu.VMEM((B,), f32),
        pltpu.VMEM((B,), f32),
        pltpu.VMEM((B, TW), f32),
        pltpu.VMEM((B, TW), f32),
        pltpu.VMEM((B,), f32),
        pltpu.VMEM((B,), f32),
        pltpu.VMEM((B,), f32),
        pltpu.SemaphoreType.DMA,
        pltpu.SemaphoreType.DMA,
        pltpu.SemaphoreType.DMA,
        pltpu.SemaphoreType.DMA,
    ]
    run = pl.kernel(
        _sc_kernel, mesh=mesh,
        out_type=(jax.ShapeDtypeStruct((N,), f32),) * 3,
        compiler_params=pltpu.CompilerParams(
            needs_layout_passes=False, use_tc_tiling_on_sc=False),
        scratch_types=per_set + per_set + [
            pltpu.HBM((NC, G_PAD, TW), f32),
            pltpu.HBM((NC, P_PAD, TW), f32),
            pltpu.VMEM((GB, TW), f32),
            pltpu.VMEM((112,), f32),
            pltpu.VMEM((16,), f32),
            pltpu.VMEM((16,), f32),
            pltpu.VMEM((16,), f32),
            pltpu.VMEM((16,), f32),
            pltpu.VMEM((16,), f32),
            pltpu.VMEM((16,), f32),
            pltpu.VMEM((16,), f32),
            pltpu.SemaphoreType.DMA,
        ] + [pltpu.VMEM((GB,), f32)] * 8,
    )
    ox, oy, oz = run(
        feature_undist[:, 0], feature_undist[:, 1], feature_undist[:, 2],
        gidx, midx, point_indices, scales_flat, relf, *gcomps, *pcomps)
    return jnp.stack([ox, oy, oz], axis=1)
